# Initial kernel scaffold; baseline (speedup 1.0000x reference)
import jax
import jax.numpy as jnp
from jax.experimental import pallas as pl


def kernel(x, edge_index, W1, b1, W2, b2):
    raise NotImplementedError("write your pallas kernel here")


# speedup vs baseline: 10.9315x; 10.9315x over previous
import functools

import jax
import jax.numpy as jnp
from jax import lax
from jax.experimental import pallas as pl
from jax.experimental.pallas import tpu as pltpu
from jax.experimental.pallas import tpu_sc as plsc

N = 10000
NP = 10240
E = 320000
CH = 128
NCHT = 80
NW = 32
EP = NW * NCHT * CH
ROWS_PER_TILE = NP // 16


def _sc_mesh():
    return plsc.VectorSubcoreMesh(core_axis_name="c", subcore_axis_name="s")


def _make_deg():
    @functools.partial(
        pl.kernel,
        out_type=(
            jax.ShapeDtypeStruct((NP,), jnp.float32),
            jax.ShapeDtypeStruct((NP,), jnp.float32),
        ),
        mesh=_sc_mesh(),
        scratch_types=[
            pltpu.VMEM((NCHT, CH), jnp.int32),
            pltpu.VMEM((CH,), jnp.float32),
            pltpu.VMEM((ROWS_PER_TILE,), jnp.float32),
            pltpu.VMEM_SHARED((NP,), jnp.float32),
        ],
    )
    def deg_kernel(dst2d, out0, out1, didx, ones_v, zbuf, acc):
        c = lax.axis_index("c")
        s = lax.axis_index("s")
        wid = s * 2 + c
        base = s * ROWS_PER_TILE

        pltpu.sync_copy(dst2d.at[pl.ds(wid * NCHT, NCHT)], didx)
        for i in range(CH // 16):
            ones_v[pl.ds(i * 16, 16)] = jnp.ones((16,), jnp.float32)

        def zfill(i, carry):
            zbuf[pl.ds(i * 16, 16)] = jnp.zeros((16,), jnp.float32)
            return carry

        lax.fori_loop(0, ROWS_PER_TILE // 16, zfill, 0)
        pltpu.sync_copy(zbuf, acc.at[pl.ds(base, ROWS_PER_TILE)])
        plsc.subcore_barrier()

        def body(j, carry):
            pltpu.sync_copy(ones_v, acc.at[didx.at[j]], add=True)
            return carry

        lax.fori_loop(0, NCHT, body, 0)
        plsc.subcore_barrier()

        @pl.when(c == 0)
        def _():
            pltpu.sync_copy(acc.at[pl.ds(base, ROWS_PER_TILE)],
                            out0.at[pl.ds(base, ROWS_PER_TILE)])

        @pl.when(c == 1)
        def _():
            pltpu.sync_copy(acc.at[pl.ds(base, ROWS_PER_TILE)],
                            out1.at[pl.ds(base, ROWS_PER_TILE)])

    return deg_kernel


def _make_edge_scatter(d):
    @functools.partial(
        pl.kernel,
        out_type=(
            jax.ShapeDtypeStruct((NP, d), jnp.float32),
            jax.ShapeDtypeStruct((NP, d), jnp.float32),
        ),
        mesh=_sc_mesh(),
        scratch_types=[
            pltpu.VMEM((NCHT, CH), jnp.int32),
            pltpu.VMEM((NCHT, CH), jnp.int32),
            pltpu.VMEM((CH, d), jnp.float32),
            pltpu.VMEM_SHARED((NP, d), jnp.float32),
        ],
        compiler_params=pltpu.CompilerParams(use_tc_tiling_on_sc=(d % 128 == 0)),
    )
    def edge_kernel(table, src2d, dst2d, out0, out1,
                    sidx, didx, rows0, acc):
        c = lax.axis_index("c")
        s = lax.axis_index("s")
        wid = s * 2 + c
        base = s * ROWS_PER_TILE

        pltpu.sync_copy(src2d.at[pl.ds(wid * NCHT, NCHT)], sidx)
        pltpu.sync_copy(dst2d.at[pl.ds(wid * NCHT, NCHT)], didx)
        pltpu.sync_copy(table.at[pl.ds(base, ROWS_PER_TILE)],
                        acc.at[pl.ds(base, ROWS_PER_TILE)])
        plsc.subcore_barrier()

        def body(j, carry):
            pltpu.sync_copy(table.at[sidx.at[j]], rows0)
            pltpu.sync_copy(rows0, acc.at[didx.at[j]], add=True)
            return carry

        lax.fori_loop(0, NCHT, body, 0)
        plsc.subcore_barrier()

        @pl.when(c == 0)
        def _():
            pltpu.sync_copy(acc.at[pl.ds(base, ROWS_PER_TILE)],
                            out0.at[pl.ds(base, ROWS_PER_TILE)])

        @pl.when(c == 1)
        def _():
            pltpu.sync_copy(acc.at[pl.ds(base, ROWS_PER_TILE)],
                            out1.at[pl.ds(base, ROWS_PER_TILE)])

    return edge_kernel


_make_deg = functools.cache(_make_deg)
_make_edge_scatter = functools.cache(_make_edge_scatter)

_BLK = 512


def _tc1(x_pad, W1, d0, d1):
    def body(x_ref, w_ref, d0_ref, d1_ref, h_ref, dinv_ref):
        dsum = d0_ref[...] + d1_ref[...] + 1.0
        dinv = lax.rsqrt(dsum)
        h = jnp.dot(x_ref[...], w_ref[...], preferred_element_type=jnp.float32)
        h_ref[...] = h * dinv
        dinv_ref[...] = dinv

    return pl.pallas_call(
        body,
        grid=(NP // _BLK,),
        in_specs=[
            pl.BlockSpec((_BLK, 128), lambda i: (i, 0)),
            pl.BlockSpec((128, 128), lambda i: (0, 0)),
            pl.BlockSpec((_BLK, 1), lambda i: (i, 0)),
            pl.BlockSpec((_BLK, 1), lambda i: (i, 0)),
        ],
        out_specs=[
            pl.BlockSpec((_BLK, 128), lambda i: (i, 0)),
            pl.BlockSpec((_BLK, 1), lambda i: (i, 0)),
        ],
        out_shape=[
            jax.ShapeDtypeStruct((NP, 128), jnp.float32),
            jax.ShapeDtypeStruct((NP, 1), jnp.float32),
        ],
    )(x_pad, W1, d0, d1)


def _tc2(p0, p1, hp, dinv, b1, W2):
    def body(p0_ref, p1_ref, hp_ref, dinv_ref, b1_ref, w2_ref, out_ref):
        t = dinv_ref[...] * (p0_ref[...] + p1_ref[...] - hp_ref[...]) + b1_ref[...]
        h = jnp.maximum(t, 0.0)
        out_ref[...] = dinv_ref[...] * jnp.dot(
            h, w2_ref[...], preferred_element_type=jnp.float32)

    return pl.pallas_call(
        body,
        grid=(NP // _BLK,),
        in_specs=[
            pl.BlockSpec((_BLK, 128), lambda i: (i, 0)),
            pl.BlockSpec((_BLK, 128), lambda i: (i, 0)),
            pl.BlockSpec((_BLK, 128), lambda i: (i, 0)),
            pl.BlockSpec((_BLK, 1), lambda i: (i, 0)),
            pl.BlockSpec((1, 128), lambda i: (0, 0)),
            pl.BlockSpec((128, 64), lambda i: (0, 0)),
        ],
        out_specs=pl.BlockSpec((_BLK, 64), lambda i: (i, 0)),
        out_shape=jax.ShapeDtypeStruct((NP, 64), jnp.float32),
    )(p0, p1, hp, dinv, b1, W2)


def _tc3(q0, q1, hp, dinv, b2):
    def body(q0_ref, q1_ref, hp_ref, dinv_ref, b2_ref, out_ref):
        out_ref[...] = dinv_ref[...] * (
            q0_ref[...] + q1_ref[...] - hp_ref[...]) + b2_ref[...]

    return pl.pallas_call(
        body,
        grid=(NP // _BLK,),
        in_specs=[
            pl.BlockSpec((_BLK, 64), lambda i: (i, 0)),
            pl.BlockSpec((_BLK, 64), lambda i: (i, 0)),
            pl.BlockSpec((_BLK, 64), lambda i: (i, 0)),
            pl.BlockSpec((_BLK, 1), lambda i: (i, 0)),
            pl.BlockSpec((1, 64), lambda i: (0, 0)),
        ],
        out_specs=pl.BlockSpec((_BLK, 64), lambda i: (i, 0)),
        out_shape=jax.ShapeDtypeStruct((NP, 64), jnp.float32),
    )(q0, q1, hp, dinv, b2)


def kernel(x, edge_index, W1, b1, W2, b2):
    pad_e = EP - E
    pad_idx = jnp.full((pad_e,), NP - 1, dtype=jnp.int32)
    src2d = jnp.concatenate([edge_index[0], pad_idx]).reshape(NW * NCHT, CH)
    dst2d = jnp.concatenate([edge_index[1], pad_idx]).reshape(NW * NCHT, CH)
    x_pad = jnp.pad(x, ((0, NP - N), (0, 0)))

    d0, d1 = _make_deg()(dst2d)
    h1p, dinv = _tc1(x_pad, W1, d0.reshape(NP, 1), d1.reshape(NP, 1))
    p0, p1 = _make_edge_scatter(128)(h1p, src2d, dst2d)
    h2p = _tc2(p0, p1, h1p, dinv, b1.reshape(1, 128), W2)
    q0, q1 = _make_edge_scatter(64)(h2p, src2d, dst2d)
    z = _tc3(q0, q1, h2p, dinv, b2.reshape(1, 64))
    return z[:N]


# speedup vs baseline: 11.7975x; 1.0792x over previous
import functools

import jax
import jax.numpy as jnp
from jax import lax
from jax.experimental import pallas as pl
from jax.experimental.pallas import tpu as pltpu
from jax.experimental.pallas import tpu_sc as plsc

N = 10000
NP = 10240
E = 320000
CH = 128
NCHT = 80
NB = 40
NW = 32
EP = NW * NCHT * CH
ROWS_PER_TILE = NP // 16


def _sc_mesh():
    return plsc.VectorSubcoreMesh(core_axis_name="c", subcore_axis_name="s")


def _make_deg():
    @functools.partial(
        pl.kernel,
        out_type=(
            jax.ShapeDtypeStruct((NP,), jnp.float32),
            jax.ShapeDtypeStruct((NP,), jnp.float32),
        ),
        mesh=_sc_mesh(),
        scratch_types=[
            pltpu.VMEM((NCHT, CH), jnp.int32),
            pltpu.VMEM((CH,), jnp.float32),
            pltpu.VMEM((ROWS_PER_TILE,), jnp.float32),
            pltpu.VMEM_SHARED((NP,), jnp.float32),
        ],
    )
    def deg_kernel(dst2d, out0, out1, didx, ones_v, zbuf, acc):
        c = lax.axis_index("c")
        s = lax.axis_index("s")
        wid = s * 2 + c
        base = s * ROWS_PER_TILE

        pltpu.sync_copy(dst2d.at[pl.ds(wid * NCHT, NCHT)], didx)
        for i in range(CH // 16):
            ones_v[pl.ds(i * 16, 16)] = jnp.ones((16,), jnp.float32)

        def zfill(i, carry):
            zbuf[pl.ds(i * 16, 16)] = jnp.zeros((16,), jnp.float32)
            return carry

        lax.fori_loop(0, ROWS_PER_TILE // 16, zfill, 0)
        pltpu.sync_copy(zbuf, acc.at[pl.ds(base, ROWS_PER_TILE)])
        plsc.subcore_barrier()

        def body(j, carry):
            pltpu.sync_copy(ones_v, acc.at[didx.at[j]], add=True)
            return carry

        lax.fori_loop(0, NCHT, body, 0)
        plsc.subcore_barrier()

        @pl.when(c == 0)
        def _():
            pltpu.sync_copy(acc.at[pl.ds(base, ROWS_PER_TILE)],
                            out0.at[pl.ds(base, ROWS_PER_TILE)])

        @pl.when(c == 1)
        def _():
            pltpu.sync_copy(acc.at[pl.ds(base, ROWS_PER_TILE)],
                            out1.at[pl.ds(base, ROWS_PER_TILE)])

    return deg_kernel


def _make_edge_scatter(d):
    @functools.partial(
        pl.kernel,
        out_type=(
            jax.ShapeDtypeStruct((NP, d), jnp.float32),
            jax.ShapeDtypeStruct((NP, d), jnp.float32),
        ),
        mesh=_sc_mesh(),
        scratch_types=[
            pltpu.VMEM((NB, CH), jnp.int32),
            pltpu.VMEM((NB, CH), jnp.int32),
            pltpu.VMEM((CH, d), jnp.float32),
            pltpu.VMEM((CH, d), jnp.float32),
            pltpu.VMEM_SHARED((NP, d), jnp.float32),
            pltpu.SemaphoreType.DMA,
            pltpu.SemaphoreType.DMA,
        ],
        compiler_params=pltpu.CompilerParams(use_tc_tiling_on_sc=(d % 128 == 0)),
    )
    def edge_kernel(table, src2d, dst2d, out0, out1,
                    sidx, didx, rows0, rows1, acc, sem0, sem1):
        c = lax.axis_index("c")
        s = lax.axis_index("s")
        wid = s * 2 + c
        base = s * ROWS_PER_TILE

        pltpu.sync_copy(table.at[pl.ds(base, ROWS_PER_TILE)],
                        acc.at[pl.ds(base, ROWS_PER_TILE)])
        plsc.subcore_barrier()

        def gather(j, rbuf, sem):
            pltpu.async_copy(table.at[sidx.at[j]], rbuf, sem)

        def wait(rbuf, sem):
            pltpu.make_async_copy(table.at[sidx.at[0]], rbuf, sem).wait()

        def scat(j, rbuf):
            pltpu.sync_copy(rbuf, acc.at[didx.at[j]], add=True)

        def block(blk, carry):
            ch0 = wid * NCHT + blk * NB
            pltpu.sync_copy(src2d.at[pl.ds(ch0, NB)], sidx)
            pltpu.sync_copy(dst2d.at[pl.ds(ch0, NB)], didx)
            gather(0, rows0, sem0)

            def body(i, carry2):
                j = i * 2
                wait(rows0, sem0)
                gather(j + 1, rows1, sem1)
                scat(j, rows0)
                wait(rows1, sem1)
                gather(j + 2, rows0, sem0)
                scat(j + 1, rows1)
                return carry2

            lax.fori_loop(0, (NB - 2) // 2, body, 0)
            wait(rows0, sem0)
            gather(NB - 1, rows1, sem1)
            scat(NB - 2, rows0)
            wait(rows1, sem1)
            scat(NB - 1, rows1)
            return carry

        lax.fori_loop(0, NCHT // NB, block, 0)
        plsc.subcore_barrier()

        @pl.when(c == 0)
        def _():
            pltpu.sync_copy(acc.at[pl.ds(base, ROWS_PER_TILE)],
                            out0.at[pl.ds(base, ROWS_PER_TILE)])

        @pl.when(c == 1)
        def _():
            pltpu.sync_copy(acc.at[pl.ds(base, ROWS_PER_TILE)],
                            out1.at[pl.ds(base, ROWS_PER_TILE)])

    return edge_kernel


_make_deg = functools.cache(_make_deg)
_make_edge_scatter = functools.cache(_make_edge_scatter)

_BLK = 512


def _tc1(x_pad, W1, d0, d1):
    def body(x_ref, w_ref, d0_ref, d1_ref, h_ref, dinv_ref):
        dsum = d0_ref[...] + d1_ref[...] + 1.0
        dinv = lax.rsqrt(dsum)
        h = jnp.dot(x_ref[...], w_ref[...], preferred_element_type=jnp.float32)
        h_ref[...] = h * dinv
        dinv_ref[...] = dinv

    return pl.pallas_call(
        body,
        grid=(NP // _BLK,),
        in_specs=[
            pl.BlockSpec((_BLK, 128), lambda i: (i, 0)),
            pl.BlockSpec((128, 128), lambda i: (0, 0)),
            pl.BlockSpec((_BLK, 1), lambda i: (i, 0)),
            pl.BlockSpec((_BLK, 1), lambda i: (i, 0)),
        ],
        out_specs=[
            pl.BlockSpec((_BLK, 128), lambda i: (i, 0)),
            pl.BlockSpec((_BLK, 1), lambda i: (i, 0)),
        ],
        out_shape=[
            jax.ShapeDtypeStruct((NP, 128), jnp.float32),
            jax.ShapeDtypeStruct((NP, 1), jnp.float32),
        ],
    )(x_pad, W1, d0, d1)


def _tc2(p0, p1, hp, dinv, b1, W2):
    def body(p0_ref, p1_ref, hp_ref, dinv_ref, b1_ref, w2_ref, out_ref):
        t = dinv_ref[...] * (p0_ref[...] + p1_ref[...] - hp_ref[...]) + b1_ref[...]
        h = jnp.maximum(t, 0.0)
        out_ref[...] = dinv_ref[...] * jnp.dot(
            h, w2_ref[...], preferred_element_type=jnp.float32)

    return pl.pallas_call(
        body,
        grid=(NP // _BLK,),
        in_specs=[
            pl.BlockSpec((_BLK, 128), lambda i: (i, 0)),
            pl.BlockSpec((_BLK, 128), lambda i: (i, 0)),
            pl.BlockSpec((_BLK, 128), lambda i: (i, 0)),
            pl.BlockSpec((_BLK, 1), lambda i: (i, 0)),
            pl.BlockSpec((1, 128), lambda i: (0, 0)),
            pl.BlockSpec((128, 64), lambda i: (0, 0)),
        ],
        out_specs=pl.BlockSpec((_BLK, 64), lambda i: (i, 0)),
        out_shape=jax.ShapeDtypeStruct((NP, 64), jnp.float32),
    )(p0, p1, hp, dinv, b1, W2)


def _tc3(q0, q1, hp, dinv, b2):
    def body(q0_ref, q1_ref, hp_ref, dinv_ref, b2_ref, out_ref):
        out_ref[...] = dinv_ref[...] * (
            q0_ref[...] + q1_ref[...] - hp_ref[...]) + b2_ref[...]

    return pl.pallas_call(
        body,
        grid=(NP // _BLK,),
        in_specs=[
            pl.BlockSpec((_BLK, 64), lambda i: (i, 0)),
            pl.BlockSpec((_BLK, 64), lambda i: (i, 0)),
            pl.BlockSpec((_BLK, 64), lambda i: (i, 0)),
            pl.BlockSpec((_BLK, 1), lambda i: (i, 0)),
            pl.BlockSpec((1, 64), lambda i: (0, 0)),
        ],
        out_specs=pl.BlockSpec((_BLK, 64), lambda i: (i, 0)),
        out_shape=jax.ShapeDtypeStruct((NP, 64), jnp.float32),
    )(q0, q1, hp, dinv, b2)


def kernel(x, edge_index, W1, b1, W2, b2):
    pad_e = EP - E
    pad_idx = jnp.full((pad_e,), NP - 1, dtype=jnp.int32)
    src2d = jnp.concatenate([edge_index[0], pad_idx]).reshape(NW * NCHT, CH)
    dst2d = jnp.concatenate([edge_index[1], pad_idx]).reshape(NW * NCHT, CH)
    x_pad = jnp.pad(x, ((0, NP - N), (0, 0)))

    d0, d1 = _make_deg()(dst2d)
    h1p, dinv = _tc1(x_pad, W1, d0.reshape(NP, 1), d1.reshape(NP, 1))
    p0, p1 = _make_edge_scatter(128)(h1p, src2d, dst2d)
    h2p = _tc2(p0, p1, h1p, dinv, b1.reshape(1, 128), W2)
    q0, q1 = _make_edge_scatter(64)(h2p, src2d, dst2d)
    z = _tc3(q0, q1, h2p, dinv, b2.reshape(1, 64))
    return z[:N]


# speedup vs baseline: 13.0539x; 1.1065x over previous
import functools

import jax
import jax.numpy as jnp
from jax import lax
from jax.experimental import pallas as pl
from jax.experimental.pallas import tpu as pltpu
from jax.experimental.pallas import tpu_sc as plsc

N = 10000
NP = 10240
E = 320000
CH = 128
NCHT = 80
NB = 40
NBF = 3
NBS = 1
NW = 32
EP = NW * NCHT * CH
ROWS_PER_TILE = NP // 16


def _sc_mesh():
    return plsc.VectorSubcoreMesh(core_axis_name="c", subcore_axis_name="s")


def _make_deg():
    @functools.partial(
        pl.kernel,
        out_type=(
            jax.ShapeDtypeStruct((NP,), jnp.float32),
            jax.ShapeDtypeStruct((NP,), jnp.float32),
        ),
        mesh=_sc_mesh(),
        scratch_types=[
            pltpu.VMEM((NCHT, CH), jnp.int32),
            pltpu.VMEM((CH,), jnp.float32),
            pltpu.VMEM((ROWS_PER_TILE,), jnp.float32),
            pltpu.VMEM_SHARED((NP,), jnp.float32),
        ],
    )
    def deg_kernel(dst2d, out0, out1, didx, ones_v, zbuf, acc):
        c = lax.axis_index("c")
        s = lax.axis_index("s")
        wid = s * 2 + c
        base = s * ROWS_PER_TILE

        pltpu.sync_copy(dst2d.at[pl.ds(wid * NCHT, NCHT)], didx)
        for i in range(CH // 16):
            ones_v[pl.ds(i * 16, 16)] = jnp.ones((16,), jnp.float32)

        def zfill(i, carry):
            zbuf[pl.ds(i * 16, 16)] = jnp.zeros((16,), jnp.float32)
            return carry

        lax.fori_loop(0, ROWS_PER_TILE // 16, zfill, 0)
        pltpu.sync_copy(zbuf, acc.at[pl.ds(base, ROWS_PER_TILE)])
        plsc.subcore_barrier()

        def body(j, carry):
            pltpu.sync_copy(ones_v, acc.at[didx.at[j]], add=True)
            return carry

        lax.fori_loop(0, NCHT, body, 0)
        plsc.subcore_barrier()

        @pl.when(c == 0)
        def _():
            pltpu.sync_copy(acc.at[pl.ds(base, ROWS_PER_TILE)],
                            out0.at[pl.ds(base, ROWS_PER_TILE)])

        @pl.when(c == 1)
        def _():
            pltpu.sync_copy(acc.at[pl.ds(base, ROWS_PER_TILE)],
                            out1.at[pl.ds(base, ROWS_PER_TILE)])

    return deg_kernel


def _make_edge_scatter(d):
    @functools.partial(
        pl.kernel,
        out_type=(
            jax.ShapeDtypeStruct((NP, d), jnp.float32),
            jax.ShapeDtypeStruct((NP, d), jnp.float32),
        ),
        mesh=_sc_mesh(),
        scratch_types=[
            pltpu.VMEM((NB, CH), jnp.int32),
            pltpu.VMEM((NB, CH), jnp.int32),
            pltpu.VMEM((CH, d), jnp.float32),
            pltpu.VMEM((CH, d), jnp.float32),
            pltpu.VMEM_SHARED((NP, d), jnp.float32),
            pltpu.SemaphoreType.DMA,
            pltpu.SemaphoreType.DMA,
        ],
        compiler_params=pltpu.CompilerParams(use_tc_tiling_on_sc=(d % 128 == 0)),
    )
    def edge_kernel(table, src2d, dst2d, out0, out1,
                    sidx, didx, rows0, rows1, acc, sem0, sem1):
        c = lax.axis_index("c")
        s = lax.axis_index("s")
        wid = s * 2 + c
        base = s * ROWS_PER_TILE

        pltpu.sync_copy(table.at[pl.ds(base, ROWS_PER_TILE)],
                        acc.at[pl.ds(base, ROWS_PER_TILE)])
        plsc.subcore_barrier()

        def gather(j, rbuf, sem):
            pltpu.async_copy(table.at[sidx.at[j]], rbuf, sem)

        def wait(rbuf, sem):
            pltpu.make_async_copy(table.at[sidx.at[0]], rbuf, sem).wait()

        def scat(j, rbuf):
            pltpu.sync_copy(rbuf, acc.at[didx.at[j]], add=True)

        my_blocks = jnp.where(c == 0, NBF, NBS)
        my_chunk0 = jnp.where(c == 0, s * (NBF * NB), 16 * NBF * NB + s * (NBS * NB))

        def block(blk, carry):
            ch0 = my_chunk0 + blk * NB
            pltpu.sync_copy(src2d.at[pl.ds(ch0, NB)], sidx)
            pltpu.sync_copy(dst2d.at[pl.ds(ch0, NB)], didx)
            gather(0, rows0, sem0)

            def body(i, carry2):
                j = i * 2
                wait(rows0, sem0)
                gather(j + 1, rows1, sem1)
                scat(j, rows0)
                wait(rows1, sem1)
                gather(j + 2, rows0, sem0)
                scat(j + 1, rows1)
                return carry2

            lax.fori_loop(0, (NB - 2) // 2, body, 0)
            wait(rows0, sem0)
            gather(NB - 1, rows1, sem1)
            scat(NB - 2, rows0)
            wait(rows1, sem1)
            scat(NB - 1, rows1)
            return carry

        lax.fori_loop(0, my_blocks, block, 0)
        plsc.subcore_barrier()

        @pl.when(c == 0)
        def _():
            pltpu.sync_copy(acc.at[pl.ds(base, ROWS_PER_TILE)],
                            out0.at[pl.ds(base, ROWS_PER_TILE)])

        @pl.when(c == 1)
        def _():
            pltpu.sync_copy(acc.at[pl.ds(base, ROWS_PER_TILE)],
                            out1.at[pl.ds(base, ROWS_PER_TILE)])

    return edge_kernel


_make_deg = functools.cache(_make_deg)
_make_edge_scatter = functools.cache(_make_edge_scatter)

_BLK = 512


def _tc1(x_pad, W1, d0, d1):
    def body(x_ref, w_ref, d0_ref, d1_ref, h_ref, dinv_ref):
        dsum = d0_ref[...] + d1_ref[...] + 1.0
        dinv = lax.rsqrt(dsum)
        h = jnp.dot(x_ref[...], w_ref[...], preferred_element_type=jnp.float32)
        h_ref[...] = h * dinv
        dinv_ref[...] = dinv

    return pl.pallas_call(
        body,
        grid=(NP // _BLK,),
        in_specs=[
            pl.BlockSpec((_BLK, 128), lambda i: (i, 0)),
            pl.BlockSpec((128, 128), lambda i: (0, 0)),
            pl.BlockSpec((_BLK, 1), lambda i: (i, 0)),
            pl.BlockSpec((_BLK, 1), lambda i: (i, 0)),
        ],
        out_specs=[
            pl.BlockSpec((_BLK, 128), lambda i: (i, 0)),
            pl.BlockSpec((_BLK, 1), lambda i: (i, 0)),
        ],
        out_shape=[
            jax.ShapeDtypeStruct((NP, 128), jnp.float32),
            jax.ShapeDtypeStruct((NP, 1), jnp.float32),
        ],
    )(x_pad, W1, d0, d1)


def _tc2(p0, p1, hp, dinv, b1, W2):
    def body(p0_ref, p1_ref, hp_ref, dinv_ref, b1_ref, w2_ref, out_ref):
        t = dinv_ref[...] * (p0_ref[...] + p1_ref[...] - hp_ref[...]) + b1_ref[...]
        h = jnp.maximum(t, 0.0)
        out_ref[...] = dinv_ref[...] * jnp.dot(
            h, w2_ref[...], preferred_element_type=jnp.float32)

    return pl.pallas_call(
        body,
        grid=(NP // _BLK,),
        in_specs=[
            pl.BlockSpec((_BLK, 128), lambda i: (i, 0)),
            pl.BlockSpec((_BLK, 128), lambda i: (i, 0)),
            pl.BlockSpec((_BLK, 128), lambda i: (i, 0)),
            pl.BlockSpec((_BLK, 1), lambda i: (i, 0)),
            pl.BlockSpec((1, 128), lambda i: (0, 0)),
            pl.BlockSpec((128, 64), lambda i: (0, 0)),
        ],
        out_specs=pl.BlockSpec((_BLK, 64), lambda i: (i, 0)),
        out_shape=jax.ShapeDtypeStruct((NP, 64), jnp.float32),
    )(p0, p1, hp, dinv, b1, W2)


def _tc3(q0, q1, hp, dinv, b2):
    def body(q0_ref, q1_ref, hp_ref, dinv_ref, b2_ref, out_ref):
        out_ref[...] = dinv_ref[...] * (
            q0_ref[...] + q1_ref[...] - hp_ref[...]) + b2_ref[...]

    return pl.pallas_call(
        body,
        grid=(NP // _BLK,),
        in_specs=[
            pl.BlockSpec((_BLK, 64), lambda i: (i, 0)),
            pl.BlockSpec((_BLK, 64), lambda i: (i, 0)),
            pl.BlockSpec((_BLK, 64), lambda i: (i, 0)),
            pl.BlockSpec((_BLK, 1), lambda i: (i, 0)),
            pl.BlockSpec((1, 64), lambda i: (0, 0)),
        ],
        out_specs=pl.BlockSpec((_BLK, 64), lambda i: (i, 0)),
        out_shape=jax.ShapeDtypeStruct((NP, 64), jnp.float32),
    )(q0, q1, hp, dinv, b2)


def kernel(x, edge_index, W1, b1, W2, b2):
    pad_e = EP - E
    pad_idx = jnp.full((pad_e,), NP - 1, dtype=jnp.int32)
    src2d = jnp.concatenate([edge_index[0], pad_idx]).reshape(NW * NCHT, CH)
    dst2d = jnp.concatenate([edge_index[1], pad_idx]).reshape(NW * NCHT, CH)
    x_pad = jnp.pad(x, ((0, NP - N), (0, 0)))

    d0, d1 = _make_deg()(dst2d)
    h1p, dinv = _tc1(x_pad, W1, d0.reshape(NP, 1), d1.reshape(NP, 1))
    p0, p1 = _make_edge_scatter(128)(h1p, src2d, dst2d)
    h2p = _tc2(p0, p1, h1p, dinv, b1.reshape(1, 128), W2)
    q0, q1 = _make_edge_scatter(64)(h2p, src2d, dst2d)
    z = _tc3(q0, q1, h2p, dinv, b2.reshape(1, 64))
    return z[:N]


# speedup vs baseline: 13.6762x; 1.0477x over previous
import functools

import jax
import jax.numpy as jnp
from jax import lax
from jax.experimental import pallas as pl
from jax.experimental.pallas import tpu as pltpu
from jax.experimental.pallas import tpu_sc as plsc

N = 10000
NP = 10240
E = 320000
CH = 128
NCHT = 80
NB = 40
NBF = 3
NBS = 1
NW = 32
EP = NW * NCHT * CH
ROWS_PER_TILE = NP // 16


def _sc_mesh():
    return plsc.VectorSubcoreMesh(core_axis_name="c", subcore_axis_name="s")


def _make_deg():
    @functools.partial(
        pl.kernel,
        out_type=(
            jax.ShapeDtypeStruct((NP,), jnp.float32),
            jax.ShapeDtypeStruct((NP,), jnp.float32),
        ),
        mesh=_sc_mesh(),
        scratch_types=[
            pltpu.VMEM((NCHT, CH), jnp.int32),
            pltpu.VMEM((CH,), jnp.float32),
            pltpu.VMEM((ROWS_PER_TILE,), jnp.float32),
            pltpu.VMEM_SHARED((NP,), jnp.float32),
        ],
    )
    def deg_kernel(dst2d, out0, out1, didx, ones_v, zbuf, acc):
        c = lax.axis_index("c")
        s = lax.axis_index("s")
        wid = s * 2 + c
        base = s * ROWS_PER_TILE

        pltpu.sync_copy(dst2d.at[pl.ds(wid * NCHT, NCHT)], didx)
        for i in range(CH // 16):
            ones_v[pl.ds(i * 16, 16)] = jnp.ones((16,), jnp.float32)

        def zfill(i, carry):
            zbuf[pl.ds(i * 16, 16)] = jnp.zeros((16,), jnp.float32)
            return carry

        lax.fori_loop(0, ROWS_PER_TILE // 16, zfill, 0)
        pltpu.sync_copy(zbuf, acc.at[pl.ds(base, ROWS_PER_TILE)])
        plsc.subcore_barrier()

        def body(j, carry):
            pltpu.sync_copy(ones_v, acc.at[didx.at[j]], add=True)
            return carry

        lax.fori_loop(0, NCHT, body, 0)
        plsc.subcore_barrier()

        @pl.when(c == 0)
        def _():
            pltpu.sync_copy(acc.at[pl.ds(base, ROWS_PER_TILE)],
                            out0.at[pl.ds(base, ROWS_PER_TILE)])

        @pl.when(c == 1)
        def _():
            pltpu.sync_copy(acc.at[pl.ds(base, ROWS_PER_TILE)],
                            out1.at[pl.ds(base, ROWS_PER_TILE)])

    return deg_kernel


def _make_edge_scatter(d):
    @functools.partial(
        pl.kernel,
        out_type=(
            jax.ShapeDtypeStruct((NP, d), jnp.float32),
            jax.ShapeDtypeStruct((NP, d), jnp.float32),
        ),
        mesh=_sc_mesh(),
        scratch_types=[
            pltpu.VMEM((NB, CH), jnp.int32),
            pltpu.VMEM((NB, CH), jnp.int32),
            pltpu.VMEM((CH, d), jnp.float32),
            pltpu.VMEM((CH, d), jnp.float32),
            pltpu.VMEM_SHARED((NP, d), jnp.float32),
            pltpu.SemaphoreType.DMA,
            pltpu.SemaphoreType.DMA,
        ],
        compiler_params=pltpu.CompilerParams(use_tc_tiling_on_sc=(d % 128 == 0)),
    )
    def edge_kernel(table, src2d, dst2d, out0, out1,
                    sidx, didx, rows0, rows1, acc, sem0, sem1):
        c = lax.axis_index("c")
        s = lax.axis_index("s")
        wid = s * 2 + c
        base = s * ROWS_PER_TILE

        pltpu.sync_copy(table.at[pl.ds(base, ROWS_PER_TILE)],
                        acc.at[pl.ds(base, ROWS_PER_TILE)])
        plsc.subcore_barrier()

        def gather(j, rbuf, sem):
            pltpu.async_copy(table.at[sidx.at[j]], rbuf, sem)

        def wait(rbuf, sem):
            pltpu.make_async_copy(table.at[sidx.at[0]], rbuf, sem).wait()

        def scat(j, rbuf):
            pltpu.sync_copy(rbuf, acc.at[didx.at[j]], add=True)

        my_blocks = jnp.where(c == 1, NBF, NBS)
        my_chunk0 = jnp.where(c == 1, s * (NBF * NB), 16 * NBF * NB + s * (NBS * NB))

        def block(blk, carry):
            ch0 = my_chunk0 + blk * NB
            pltpu.sync_copy(src2d.at[pl.ds(ch0, NB)], sidx)
            pltpu.sync_copy(dst2d.at[pl.ds(ch0, NB)], didx)
            gather(0, rows0, sem0)

            def body(i, carry2):
                j = i * 2
                wait(rows0, sem0)
                gather(j + 1, rows1, sem1)
                scat(j, rows0)
                wait(rows1, sem1)
                gather(j + 2, rows0, sem0)
                scat(j + 1, rows1)
                return carry2

            lax.fori_loop(0, (NB - 2) // 2, body, 0)
            wait(rows0, sem0)
            gather(NB - 1, rows1, sem1)
            scat(NB - 2, rows0)
            wait(rows1, sem1)
            scat(NB - 1, rows1)
            return carry

        lax.fori_loop(0, my_blocks, block, 0)
        plsc.subcore_barrier()

        @pl.when(c == 0)
        def _():
            pltpu.sync_copy(acc.at[pl.ds(base, ROWS_PER_TILE)],
                            out0.at[pl.ds(base, ROWS_PER_TILE)])

        @pl.when(c == 1)
        def _():
            pltpu.sync_copy(acc.at[pl.ds(base, ROWS_PER_TILE)],
                            out1.at[pl.ds(base, ROWS_PER_TILE)])

    return edge_kernel


_make_deg = functools.cache(_make_deg)
_make_edge_scatter = functools.cache(_make_edge_scatter)

_BLK = 512


def _tc1(x_pad, W1, d0, d1):
    def body(x_ref, w_ref, d0_ref, d1_ref, h_ref, dinv_ref):
        dsum = d0_ref[...] + d1_ref[...] + 1.0
        dinv = lax.rsqrt(dsum)
        h = jnp.dot(x_ref[...], w_ref[...], preferred_element_type=jnp.float32)
        h_ref[...] = h * dinv
        dinv_ref[...] = dinv

    return pl.pallas_call(
        body,
        grid=(NP // _BLK,),
        in_specs=[
            pl.BlockSpec((_BLK, 128), lambda i: (i, 0)),
            pl.BlockSpec((128, 128), lambda i: (0, 0)),
            pl.BlockSpec((_BLK, 1), lambda i: (i, 0)),
            pl.BlockSpec((_BLK, 1), lambda i: (i, 0)),
        ],
        out_specs=[
            pl.BlockSpec((_BLK, 128), lambda i: (i, 0)),
            pl.BlockSpec((_BLK, 1), lambda i: (i, 0)),
        ],
        out_shape=[
            jax.ShapeDtypeStruct((NP, 128), jnp.float32),
            jax.ShapeDtypeStruct((NP, 1), jnp.float32),
        ],
    )(x_pad, W1, d0, d1)


def _tc2(p0, p1, hp, dinv, b1, W2):
    def body(p0_ref, p1_ref, hp_ref, dinv_ref, b1_ref, w2_ref, out_ref):
        t = dinv_ref[...] * (p0_ref[...] + p1_ref[...] - hp_ref[...]) + b1_ref[...]
        h = jnp.maximum(t, 0.0)
        out_ref[...] = dinv_ref[...] * jnp.dot(
            h, w2_ref[...], preferred_element_type=jnp.float32)

    return pl.pallas_call(
        body,
        grid=(NP // _BLK,),
        in_specs=[
            pl.BlockSpec((_BLK, 128), lambda i: (i, 0)),
            pl.BlockSpec((_BLK, 128), lambda i: (i, 0)),
            pl.BlockSpec((_BLK, 128), lambda i: (i, 0)),
            pl.BlockSpec((_BLK, 1), lambda i: (i, 0)),
            pl.BlockSpec((1, 128), lambda i: (0, 0)),
            pl.BlockSpec((128, 64), lambda i: (0, 0)),
        ],
        out_specs=pl.BlockSpec((_BLK, 64), lambda i: (i, 0)),
        out_shape=jax.ShapeDtypeStruct((NP, 64), jnp.float32),
    )(p0, p1, hp, dinv, b1, W2)


def _tc3(q0, q1, hp, dinv, b2):
    def body(q0_ref, q1_ref, hp_ref, dinv_ref, b2_ref, out_ref):
        out_ref[...] = dinv_ref[...] * (
            q0_ref[...] + q1_ref[...] - hp_ref[...]) + b2_ref[...]

    return pl.pallas_call(
        body,
        grid=(NP // _BLK,),
        in_specs=[
            pl.BlockSpec((_BLK, 64), lambda i: (i, 0)),
            pl.BlockSpec((_BLK, 64), lambda i: (i, 0)),
            pl.BlockSpec((_BLK, 64), lambda i: (i, 0)),
            pl.BlockSpec((_BLK, 1), lambda i: (i, 0)),
            pl.BlockSpec((1, 64), lambda i: (0, 0)),
        ],
        out_specs=pl.BlockSpec((_BLK, 64), lambda i: (i, 0)),
        out_shape=jax.ShapeDtypeStruct((NP, 64), jnp.float32),
    )(q0, q1, hp, dinv, b2)


def kernel(x, edge_index, W1, b1, W2, b2):
    pad_e = EP - E
    pad_idx = jnp.full((pad_e,), NP - 1, dtype=jnp.int32)
    src2d = jnp.concatenate([edge_index[0], pad_idx]).reshape(NW * NCHT, CH)
    dst2d = jnp.concatenate([edge_index[1], pad_idx]).reshape(NW * NCHT, CH)
    x_pad = jnp.pad(x, ((0, NP - N), (0, 0)))

    d0, d1 = _make_deg()(dst2d)
    h1p, dinv = _tc1(x_pad, W1, d0.reshape(NP, 1), d1.reshape(NP, 1))
    p0, p1 = _make_edge_scatter(128)(h1p, src2d, dst2d)
    h2p = _tc2(p0, p1, h1p, dinv, b1.reshape(1, 128), W2)
    q0, q1 = _make_edge_scatter(64)(h2p, src2d, dst2d)
    z = _tc3(q0, q1, h2p, dinv, b2.reshape(1, 64))
    return z[:N]


# speedup vs baseline: 16.8628x; 1.2330x over previous
import functools

import jax
import jax.numpy as jnp
from jax import lax
from jax.experimental import pallas as pl
from jax.experimental.pallas import tpu as pltpu
from jax.experimental.pallas import tpu_sc as plsc

N = 10000
NP = 10240
E = 320000
CH = 128
NCHT = 80
NB = 40
NBF = 3
NBS = 1
NW = 32
EP = NW * NCHT * CH
ROWS_PER_TILE = NP // 16


def _sc_mesh():
    return plsc.VectorSubcoreMesh(core_axis_name="c", subcore_axis_name="s")


def _make_deg():
    @functools.partial(
        pl.kernel,
        out_type=(
            jax.ShapeDtypeStruct((NP,), jnp.float32),
            jax.ShapeDtypeStruct((NP,), jnp.float32),
        ),
        mesh=_sc_mesh(),
        scratch_types=[
            pltpu.VMEM((NCHT, CH), jnp.int32),
            pltpu.VMEM((CH,), jnp.float32),
            pltpu.VMEM((ROWS_PER_TILE,), jnp.float32),
            pltpu.VMEM_SHARED((NP,), jnp.float32),
        ],
    )
    def deg_kernel(dst2d, out0, out1, didx, ones_v, zbuf, acc):
        c = lax.axis_index("c")
        s = lax.axis_index("s")
        wid = s * 2 + c
        base = s * ROWS_PER_TILE

        pltpu.sync_copy(dst2d.at[pl.ds(wid * NCHT, NCHT)], didx)
        for i in range(CH // 16):
            ones_v[pl.ds(i * 16, 16)] = jnp.ones((16,), jnp.float32)

        def zfill(i, carry):
            zbuf[pl.ds(i * 16, 16)] = jnp.zeros((16,), jnp.float32)
            return carry

        lax.fori_loop(0, ROWS_PER_TILE // 16, zfill, 0)
        pltpu.sync_copy(zbuf, acc.at[pl.ds(base, ROWS_PER_TILE)])
        plsc.subcore_barrier()

        def body(j, carry):
            pltpu.sync_copy(ones_v, acc.at[didx.at[j]], add=True)
            return carry

        lax.fori_loop(0, NCHT, body, 0)
        plsc.subcore_barrier()

        @pl.when(c == 0)
        def _():
            pltpu.sync_copy(acc.at[pl.ds(base, ROWS_PER_TILE)],
                            out0.at[pl.ds(base, ROWS_PER_TILE)])

        @pl.when(c == 1)
        def _():
            pltpu.sync_copy(acc.at[pl.ds(base, ROWS_PER_TILE)],
                            out1.at[pl.ds(base, ROWS_PER_TILE)])

    return deg_kernel


def _make_edge_scatter(d, local_table):
    scratch = [
        pltpu.VMEM((NB, CH), jnp.int32),
        pltpu.VMEM((NB, CH), jnp.int32),
        pltpu.VMEM((CH, d), jnp.float32),
        pltpu.VMEM((CH, d), jnp.float32),
        pltpu.VMEM_SHARED((NP, d), jnp.float32),
        pltpu.SemaphoreType.DMA,
        pltpu.SemaphoreType.DMA,
        pltpu.VMEM_SHARED((NP if local_table else 8, d), jnp.float32),
    ]

    @functools.partial(
        pl.kernel,
        out_type=(
            jax.ShapeDtypeStruct((NP, d), jnp.float32),
            jax.ShapeDtypeStruct((NP, d), jnp.float32),
        ),
        mesh=_sc_mesh(),
        scratch_types=scratch,
        compiler_params=pltpu.CompilerParams(use_tc_tiling_on_sc=(d % 128 == 0)),
    )
    def edge_kernel(table, src2d, dst2d, out0, out1,
                    sidx, didx, rows0, rows1, acc, sem0, sem1, table_sh):
        c = lax.axis_index("c")
        s = lax.axis_index("s")
        base = s * ROWS_PER_TILE

        pltpu.sync_copy(table.at[pl.ds(base, ROWS_PER_TILE)],
                        acc.at[pl.ds(base, ROWS_PER_TILE)])
        if local_table:
            pltpu.sync_copy(table.at[pl.ds(base, ROWS_PER_TILE)],
                            table_sh.at[pl.ds(base, ROWS_PER_TILE)])
            gsrc = table_sh
        else:
            gsrc = table
        plsc.subcore_barrier()

        def gather(j, rbuf, sem):
            pltpu.async_copy(gsrc.at[sidx.at[j]], rbuf, sem)

        def wait(rbuf, sem):
            pltpu.make_async_copy(gsrc.at[sidx.at[0]], rbuf, sem).wait()

        def scat(j, rbuf):
            pltpu.sync_copy(rbuf, acc.at[didx.at[j]], add=True)

        if local_table:
            my_blocks = 2
            my_chunk0 = (s * 2 + c) * (2 * NB)
        else:
            my_blocks = jnp.where(c == 1, NBF, NBS)
            my_chunk0 = jnp.where(c == 1, s * (NBF * NB),
                                  16 * NBF * NB + s * (NBS * NB))

        def block(blk, carry):
            ch0 = my_chunk0 + blk * NB
            pltpu.sync_copy(src2d.at[pl.ds(ch0, NB)], sidx)
            pltpu.sync_copy(dst2d.at[pl.ds(ch0, NB)], didx)
            gather(0, rows0, sem0)

            def body(i, carry2):
                j = i * 2
                wait(rows0, sem0)
                gather(j + 1, rows1, sem1)
                scat(j, rows0)
                wait(rows1, sem1)
                gather(j + 2, rows0, sem0)
                scat(j + 1, rows1)
                return carry2

            lax.fori_loop(0, (NB - 2) // 2, body, 0)
            wait(rows0, sem0)
            gather(NB - 1, rows1, sem1)
            scat(NB - 2, rows0)
            wait(rows1, sem1)
            scat(NB - 1, rows1)
            return carry

        lax.fori_loop(0, my_blocks, block, 0)
        plsc.subcore_barrier()

        @pl.when(c == 0)
        def _():
            pltpu.sync_copy(acc.at[pl.ds(base, ROWS_PER_TILE)],
                            out0.at[pl.ds(base, ROWS_PER_TILE)])

        @pl.when(c == 1)
        def _():
            pltpu.sync_copy(acc.at[pl.ds(base, ROWS_PER_TILE)],
                            out1.at[pl.ds(base, ROWS_PER_TILE)])

    return edge_kernel


_make_deg = functools.cache(_make_deg)
_make_edge_scatter = functools.cache(_make_edge_scatter)

_BLK = 512


def _tc1(x_pad, W1, d0, d1):
    def body(x_ref, w_ref, d0_ref, d1_ref, h_ref, dinv_ref):
        dsum = d0_ref[...] + d1_ref[...] + 1.0
        dinv = lax.rsqrt(dsum)
        h = jnp.dot(x_ref[...], w_ref[...], preferred_element_type=jnp.float32)
        h_ref[...] = h * dinv
        dinv_ref[...] = dinv

    return pl.pallas_call(
        body,
        grid=(NP // _BLK,),
        in_specs=[
            pl.BlockSpec((_BLK, 128), lambda i: (i, 0)),
            pl.BlockSpec((128, 128), lambda i: (0, 0)),
            pl.BlockSpec((_BLK, 1), lambda i: (i, 0)),
            pl.BlockSpec((_BLK, 1), lambda i: (i, 0)),
        ],
        out_specs=[
            pl.BlockSpec((_BLK, 128), lambda i: (i, 0)),
            pl.BlockSpec((_BLK, 1), lambda i: (i, 0)),
        ],
        out_shape=[
            jax.ShapeDtypeStruct((NP, 128), jnp.float32),
            jax.ShapeDtypeStruct((NP, 1), jnp.float32),
        ],
    )(x_pad, W1, d0, d1)


def _tc2(p0, p1, hp, dinv, b1, W2):
    def body(p0_ref, p1_ref, hp_ref, dinv_ref, b1_ref, w2_ref, out_ref):
        t = dinv_ref[...] * (p0_ref[...] + p1_ref[...] - hp_ref[...]) + b1_ref[...]
        h = jnp.maximum(t, 0.0)
        out_ref[...] = dinv_ref[...] * jnp.dot(
            h, w2_ref[...], preferred_element_type=jnp.float32)

    return pl.pallas_call(
        body,
        grid=(NP // _BLK,),
        in_specs=[
            pl.BlockSpec((_BLK, 128), lambda i: (i, 0)),
            pl.BlockSpec((_BLK, 128), lambda i: (i, 0)),
            pl.BlockSpec((_BLK, 128), lambda i: (i, 0)),
            pl.BlockSpec((_BLK, 1), lambda i: (i, 0)),
            pl.BlockSpec((1, 128), lambda i: (0, 0)),
            pl.BlockSpec((128, 64), lambda i: (0, 0)),
        ],
        out_specs=pl.BlockSpec((_BLK, 64), lambda i: (i, 0)),
        out_shape=jax.ShapeDtypeStruct((NP, 64), jnp.float32),
    )(p0, p1, hp, dinv, b1, W2)


def _tc3(q0, q1, hp, dinv, b2):
    def body(q0_ref, q1_ref, hp_ref, dinv_ref, b2_ref, out_ref):
        out_ref[...] = dinv_ref[...] * (
            q0_ref[...] + q1_ref[...] - hp_ref[...]) + b2_ref[...]

    return pl.pallas_call(
        body,
        grid=(NP // _BLK,),
        in_specs=[
            pl.BlockSpec((_BLK, 64), lambda i: (i, 0)),
            pl.BlockSpec((_BLK, 64), lambda i: (i, 0)),
            pl.BlockSpec((_BLK, 64), lambda i: (i, 0)),
            pl.BlockSpec((_BLK, 1), lambda i: (i, 0)),
            pl.BlockSpec((1, 64), lambda i: (0, 0)),
        ],
        out_specs=pl.BlockSpec((_BLK, 64), lambda i: (i, 0)),
        out_shape=jax.ShapeDtypeStruct((NP, 64), jnp.float32),
    )(q0, q1, hp, dinv, b2)


def kernel(x, edge_index, W1, b1, W2, b2):
    pad_e = EP - E
    pad_idx = jnp.full((pad_e,), NP - 1, dtype=jnp.int32)
    src2d = jnp.concatenate([edge_index[0], pad_idx]).reshape(NW * NCHT, CH)
    dst2d = jnp.concatenate([edge_index[1], pad_idx]).reshape(NW * NCHT, CH)
    x_pad = jnp.pad(x, ((0, NP - N), (0, 0)))

    d0, d1 = _make_deg()(dst2d)
    h1p, dinv = _tc1(x_pad, W1, d0.reshape(NP, 1), d1.reshape(NP, 1))
    p0, p1 = _make_edge_scatter(128, False)(h1p, src2d, dst2d)
    h2p = _tc2(p0, p1, h1p, dinv, b1.reshape(1, 128), W2)
    q0, q1 = _make_edge_scatter(64, True)(h2p, src2d, dst2d)
    z = _tc3(q0, q1, h2p, dinv, b2.reshape(1, 64))
    return z[:N]


# speedup vs baseline: 24.5788x; 1.4576x over previous
import functools

import jax
import jax.numpy as jnp
from jax import lax
from jax.experimental import pallas as pl
from jax.experimental.pallas import tpu as pltpu
from jax.experimental.pallas import tpu_sc as plsc

N = 10000
NP = 10240
E = 320000
CH = 128
NCHT = 80
NB = 40
DH = 64
NW = 32
EP = NW * NCHT * CH
ROWS_PER_TILE = NP // 16


def _sc_mesh():
    return plsc.VectorSubcoreMesh(core_axis_name="c", subcore_axis_name="s")


def _make_deg():
    @functools.partial(
        pl.kernel,
        out_type=(
            jax.ShapeDtypeStruct((NP,), jnp.float32),
            jax.ShapeDtypeStruct((NP,), jnp.float32),
        ),
        mesh=_sc_mesh(),
        scratch_types=[
            pltpu.VMEM((NCHT, CH), jnp.int32),
            pltpu.VMEM((CH,), jnp.float32),
            pltpu.VMEM((ROWS_PER_TILE,), jnp.float32),
            pltpu.VMEM_SHARED((NP,), jnp.float32),
        ],
    )
    def deg_kernel(dst2d, out0, out1, didx, ones_v, zbuf, acc):
        c = lax.axis_index("c")
        s = lax.axis_index("s")
        wid = s * 2 + c
        base = s * ROWS_PER_TILE

        pltpu.sync_copy(dst2d.at[pl.ds(wid * NCHT, NCHT)], didx)
        for i in range(CH // 16):
            ones_v[pl.ds(i * 16, 16)] = jnp.ones((16,), jnp.float32)

        def zfill(i, carry):
            zbuf[pl.ds(i * 16, 16)] = jnp.zeros((16,), jnp.float32)
            return carry

        lax.fori_loop(0, ROWS_PER_TILE // 16, zfill, 0)
        pltpu.sync_copy(zbuf, acc.at[pl.ds(base, ROWS_PER_TILE)])
        plsc.subcore_barrier()

        def body(j, carry):
            pltpu.sync_copy(ones_v, acc.at[didx.at[j]], add=True)
            return carry

        lax.fori_loop(0, NCHT, body, 0)
        plsc.subcore_barrier()

        @pl.when(c == 0)
        def _():
            pltpu.sync_copy(acc.at[pl.ds(base, ROWS_PER_TILE)],
                            out0.at[pl.ds(base, ROWS_PER_TILE)])

        @pl.when(c == 1)
        def _():
            pltpu.sync_copy(acc.at[pl.ds(base, ROWS_PER_TILE)],
                            out1.at[pl.ds(base, ROWS_PER_TILE)])

    return deg_kernel


def _make_edge_scatter(num_tables):
    @functools.partial(
        pl.kernel,
        out_type=tuple(jax.ShapeDtypeStruct((NP, DH), jnp.float32)
                       for _ in range(2 * num_tables)),
        mesh=_sc_mesh(),
        scratch_types=[
            pltpu.VMEM((NB, CH), jnp.int32),
            pltpu.VMEM((NB, CH), jnp.int32),
            pltpu.VMEM((CH, DH), jnp.float32),
            pltpu.VMEM((CH, DH), jnp.float32),
            pltpu.VMEM_SHARED((NP, DH), jnp.float32),
            pltpu.VMEM_SHARED((NP, DH), jnp.float32),
            pltpu.SemaphoreType.DMA,
            pltpu.SemaphoreType.DMA,
        ],
        compiler_params=pltpu.CompilerParams(use_tc_tiling_on_sc=False),
    )
    def edge_kernel(*refs):
        tables = refs[:num_tables]
        src2d = refs[num_tables]
        dst2d = refs[num_tables + 1]
        outs = refs[num_tables + 2:num_tables + 2 + 2 * num_tables]
        (sidx, didx, rows0, rows1, acc, table_sh, sem0, sem1) = \
            refs[num_tables + 2 + 2 * num_tables:]
        c = lax.axis_index("c")
        s = lax.axis_index("s")
        wid = s * 2 + c
        base = s * ROWS_PER_TILE

        def gather(j, rbuf, sem):
            pltpu.async_copy(table_sh.at[sidx.at[j]], rbuf, sem)

        def wait(rbuf, sem):
            pltpu.make_async_copy(table_sh.at[sidx.at[0]], rbuf, sem).wait()

        def scat(j, rbuf):
            pltpu.sync_copy(rbuf, acc.at[didx.at[j]], add=True)

        def block(blk, carry):
            ch0 = wid * NCHT + blk * NB
            pltpu.sync_copy(src2d.at[pl.ds(ch0, NB)], sidx)
            pltpu.sync_copy(dst2d.at[pl.ds(ch0, NB)], didx)
            gather(0, rows0, sem0)

            def body(i, carry2):
                j = i * 2
                wait(rows0, sem0)
                gather(j + 1, rows1, sem1)
                scat(j, rows0)
                wait(rows1, sem1)
                gather(j + 2, rows0, sem0)
                scat(j + 1, rows1)
                return carry2

            lax.fori_loop(0, (NB - 2) // 2, body, 0)
            wait(rows0, sem0)
            gather(NB - 1, rows1, sem1)
            scat(NB - 2, rows0)
            wait(rows1, sem1)
            scat(NB - 1, rows1)
            return carry

        for h in range(num_tables):
            table = tables[h]
            out0, out1 = outs[2 * h], outs[2 * h + 1]
            pltpu.sync_copy(table.at[pl.ds(base, ROWS_PER_TILE)],
                            acc.at[pl.ds(base, ROWS_PER_TILE)])
            pltpu.sync_copy(table.at[pl.ds(base, ROWS_PER_TILE)],
                            table_sh.at[pl.ds(base, ROWS_PER_TILE)])
            plsc.subcore_barrier()
            lax.fori_loop(0, NCHT // NB, block, 0)
            plsc.subcore_barrier()

            @pl.when(c == 0)
            def _():
                pltpu.sync_copy(acc.at[pl.ds(base, ROWS_PER_TILE)],
                                out0.at[pl.ds(base, ROWS_PER_TILE)])

            @pl.when(c == 1)
            def _():
                pltpu.sync_copy(acc.at[pl.ds(base, ROWS_PER_TILE)],
                                out1.at[pl.ds(base, ROWS_PER_TILE)])

    return edge_kernel


_make_deg = functools.cache(_make_deg)
_make_edge_scatter = functools.cache(_make_edge_scatter)

_BLK = 512


def _tc1(x_pad, W1, d0, d1):
    def body(x_ref, w_ref, d0_ref, d1_ref, ha_ref, hb_ref, dinv_ref):
        dsum = d0_ref[...] + d1_ref[...] + 1.0
        dinv = lax.rsqrt(dsum)
        h = jnp.dot(x_ref[...], w_ref[...], preferred_element_type=jnp.float32)
        h = h * dinv
        ha_ref[...] = h[:, :64]
        hb_ref[...] = h[:, 64:]
        dinv_ref[...] = dinv

    return pl.pallas_call(
        body,
        grid=(NP // _BLK,),
        in_specs=[
            pl.BlockSpec((_BLK, 128), lambda i: (i, 0)),
            pl.BlockSpec((128, 128), lambda i: (0, 0)),
            pl.BlockSpec((_BLK, 1), lambda i: (i, 0)),
            pl.BlockSpec((_BLK, 1), lambda i: (i, 0)),
        ],
        out_specs=[
            pl.BlockSpec((_BLK, 64), lambda i: (i, 0)),
            pl.BlockSpec((_BLK, 64), lambda i: (i, 0)),
            pl.BlockSpec((_BLK, 1), lambda i: (i, 0)),
        ],
        out_shape=[
            jax.ShapeDtypeStruct((NP, 64), jnp.float32),
            jax.ShapeDtypeStruct((NP, 64), jnp.float32),
            jax.ShapeDtypeStruct((NP, 1), jnp.float32),
        ],
    )(x_pad, W1, d0, d1)


def _tc2(pa0, pa1, pb0, pb1, ha, hb, dinv, b1, W2):
    def body(pa0_ref, pa1_ref, pb0_ref, pb1_ref, ha_ref, hb_ref,
             dinv_ref, b1_ref, w2_ref, out_ref):
        dinv = dinv_ref[...]
        ta = dinv * (pa0_ref[...] + pa1_ref[...] - ha_ref[...]) + b1_ref[..., :64]
        tb = dinv * (pb0_ref[...] + pb1_ref[...] - hb_ref[...]) + b1_ref[..., 64:]
        h = jnp.maximum(jnp.concatenate([ta, tb], axis=1), 0.0)
        out_ref[...] = dinv * jnp.dot(
            h, w2_ref[...], preferred_element_type=jnp.float32)

    half = pl.BlockSpec((_BLK, 64), lambda i: (i, 0))
    return pl.pallas_call(
        body,
        grid=(NP // _BLK,),
        in_specs=[
            half, half, half, half, half, half,
            pl.BlockSpec((_BLK, 1), lambda i: (i, 0)),
            pl.BlockSpec((1, 128), lambda i: (0, 0)),
            pl.BlockSpec((128, 64), lambda i: (0, 0)),
        ],
        out_specs=pl.BlockSpec((_BLK, 64), lambda i: (i, 0)),
        out_shape=jax.ShapeDtypeStruct((NP, 64), jnp.float32),
    )(pa0, pa1, pb0, pb1, ha, hb, dinv, b1, W2)


def _tc3(q0, q1, hp, dinv, b2):
    def body(q0_ref, q1_ref, hp_ref, dinv_ref, b2_ref, out_ref):
        out_ref[...] = dinv_ref[...] * (
            q0_ref[...] + q1_ref[...] - hp_ref[...]) + b2_ref[...]

    return pl.pallas_call(
        body,
        grid=(NP // _BLK,),
        in_specs=[
            pl.BlockSpec((_BLK, 64), lambda i: (i, 0)),
            pl.BlockSpec((_BLK, 64), lambda i: (i, 0)),
            pl.BlockSpec((_BLK, 64), lambda i: (i, 0)),
            pl.BlockSpec((_BLK, 1), lambda i: (i, 0)),
            pl.BlockSpec((1, 64), lambda i: (0, 0)),
        ],
        out_specs=pl.BlockSpec((_BLK, 64), lambda i: (i, 0)),
        out_shape=jax.ShapeDtypeStruct((NP, 64), jnp.float32),
    )(q0, q1, hp, dinv, b2)


def kernel(x, edge_index, W1, b1, W2, b2):
    pad_e = EP - E
    pad_idx = jnp.full((pad_e,), NP - 1, dtype=jnp.int32)
    src2d = jnp.concatenate([edge_index[0], pad_idx]).reshape(NW * NCHT, CH)
    dst2d = jnp.concatenate([edge_index[1], pad_idx]).reshape(NW * NCHT, CH)
    x_pad = jnp.pad(x, ((0, NP - N), (0, 0)))

    d0, d1 = _make_deg()(dst2d)
    h1a, h1b, dinv = _tc1(x_pad, W1, d0.reshape(NP, 1), d1.reshape(NP, 1))
    pa0, pa1, pb0, pb1 = _make_edge_scatter(2)(h1a, h1b, src2d, dst2d)
    h2p = _tc2(pa0, pa1, pb0, pb1, h1a, h1b, dinv, b1.reshape(1, 128), W2)
    q0, q1 = _make_edge_scatter(1)(h2p, src2d, dst2d)
    z = _tc3(q0, q1, h2p, dinv, b2.reshape(1, 64))
    return z[:N]
